# baseline (speedup 1.0000x reference)
import functools

import jax
import jax.numpy as jnp
from jax import lax
from jax.experimental import pallas as pl
from jax.experimental.pallas import tpu as pltpu
from jax.experimental.pallas import tpu_sc as plsc

N = 10000
D = 256
DP = 272
E = 320000
NCHUNK = 16
CHUNK = E // NCHUNK
NSEG = 25
SEG = CHUNK // NSEG
HALF = N // 2
RPT = 320
ACC_ROWS = 16 * RPT
MB = 400


def _mm_att_kernel(x_ref, w_ref, asrc_ref, adst_ref, h_ref, es_ref, ed_ref,
                   m_ref):
    mm = jnp.dot(x_ref[...], w_ref[...], preferred_element_type=jnp.float32)
    h_ref[...] = mm
    h_ref[:, D:] = jnp.ones((MB, DP - D), jnp.float32)
    hv = mm[:, :D]
    es = jnp.sum(hv * asrc_ref[...], axis=1, keepdims=True)
    ed = jnp.sum(hv * adst_ref[...], axis=1, keepdims=True)
    es_ref[...] = es
    ed_ref[...] = ed
    cur = jnp.concatenate([jnp.max(es).reshape(1, 1),
                           jnp.max(ed).reshape(1, 1)], axis=1)

    @pl.when(pl.program_id(0) == 0)
    def _():
        m_ref[...] = cur

    @pl.when(pl.program_id(0) != 0)
    def _():
        m_ref[...] = jnp.maximum(m_ref[...], cur)


def _mm_att(x, wT_pad, a_src, a_dst):
    m, k = x.shape
    return pl.pallas_call(
        _mm_att_kernel,
        grid=(m // MB,),
        in_specs=[
            pl.BlockSpec((MB, k), lambda i: (i, 0)),
            pl.BlockSpec((k, DP), lambda i: (0, 0)),
            pl.BlockSpec((1, D), lambda i: (0, 0)),
            pl.BlockSpec((1, D), lambda i: (0, 0)),
        ],
        out_specs=[
            pl.BlockSpec((MB, DP), lambda i: (i, 0)),
            pl.BlockSpec((MB, 1), lambda i: (i, 0)),
            pl.BlockSpec((MB, 1), lambda i: (i, 0)),
            pl.BlockSpec((1, 2), lambda i: (0, 0)),
        ],
        out_shape=[
            jax.ShapeDtypeStruct((m, DP), jnp.float32),
            jax.ShapeDtypeStruct((m, 1), jnp.float32),
            jax.ShapeDtypeStruct((m, 1), jnp.float32),
            jax.ShapeDtypeStruct((1, 2), jnp.float32),
        ],
    )(x, wT_pad, a_src, a_dst)


def _elu(x):
    return jnp.where(x > 0, x, jnp.exp(jnp.minimum(x, 0.0)) - 1.0)


def _self_loop_finish(acc_ref, h_ref, es_ref, ed_ref, c_ref, b_ref):
    z = es_ref[...] + ed_ref[...]
    e = jnp.where(z > 0, z, z * 0.2)
    eeself = jnp.exp(e - c_ref[0, 0])
    accf = acc_ref[...] + eeself * h_ref[...]
    return accf[:, :D] / accf[:, D:D + 1] + b_ref[...]


def _fin_mm_att_kernel(acc_ref, h1_ref, es1_ref, ed1_ref, c_ref, b_ref,
                       w_ref, asrc_ref, adst_ref,
                       h_ref, es_ref, ed_ref, m_ref):
    hprev = _elu(_self_loop_finish(acc_ref, h1_ref, es1_ref, ed1_ref,
                                   c_ref, b_ref))
    mm = jnp.dot(hprev, w_ref[...], preferred_element_type=jnp.float32)
    h_ref[...] = mm
    h_ref[:, D:] = jnp.ones((MB, DP - D), jnp.float32)
    hv = mm[:, :D]
    es = jnp.sum(hv * asrc_ref[...], axis=1, keepdims=True)
    ed = jnp.sum(hv * adst_ref[...], axis=1, keepdims=True)
    es_ref[...] = es
    ed_ref[...] = ed
    cur = jnp.concatenate([jnp.max(es).reshape(1, 1),
                           jnp.max(ed).reshape(1, 1)], axis=1)

    @pl.when(pl.program_id(0) == 0)
    def _():
        m_ref[...] = cur

    @pl.when(pl.program_id(0) != 0)
    def _():
        m_ref[...] = jnp.maximum(m_ref[...], cur)


def _fin_mm_att(acc, h1, es1, ed1, c, b, wT_pad, a_src, a_dst):
    m = acc.shape[0]
    return pl.pallas_call(
        _fin_mm_att_kernel,
        grid=(m // MB,),
        in_specs=[
            pl.BlockSpec((MB, DP), lambda i: (i, 0)),
            pl.BlockSpec((MB, DP), lambda i: (i, 0)),
            pl.BlockSpec((MB, 1), lambda i: (i, 0)),
            pl.BlockSpec((MB, 1), lambda i: (i, 0)),
            pl.BlockSpec((1, 1), lambda i: (0, 0)),
            pl.BlockSpec((1, D), lambda i: (0, 0)),
            pl.BlockSpec((D, DP), lambda i: (0, 0)),
            pl.BlockSpec((1, D), lambda i: (0, 0)),
            pl.BlockSpec((1, D), lambda i: (0, 0)),
        ],
        out_specs=[
            pl.BlockSpec((MB, DP), lambda i: (i, 0)),
            pl.BlockSpec((MB, 1), lambda i: (i, 0)),
            pl.BlockSpec((MB, 1), lambda i: (i, 0)),
            pl.BlockSpec((1, 2), lambda i: (0, 0)),
        ],
        out_shape=[
            jax.ShapeDtypeStruct((m, DP), jnp.float32),
            jax.ShapeDtypeStruct((m, 1), jnp.float32),
            jax.ShapeDtypeStruct((m, 1), jnp.float32),
            jax.ShapeDtypeStruct((1, 2), jnp.float32),
        ],
    )(acc, h1, es1, ed1, c, b, wT_pad, a_src, a_dst)


def _fin_mm_stats_kernel(acc_ref, h2_ref, es2_ref, ed2_ref, c_ref, b_ref,
                         w_ref, wb_ref, z_ref, st_ref):
    h = _elu(_self_loop_finish(acc_ref, h2_ref, es2_ref, ed2_ref,
                               c_ref, b_ref))
    z = jnp.dot(h, w_ref[...], preferred_element_type=jnp.float32) + wb_ref[...]
    z_ref[...] = z
    cur = jnp.concatenate([jnp.sum(z, axis=0, keepdims=True),
                           jnp.sum(z * z, axis=0, keepdims=True)], axis=0)

    @pl.when(pl.program_id(0) == 0)
    def _():
        st_ref[...] = cur

    @pl.when(pl.program_id(0) != 0)
    def _():
        st_ref[...] = st_ref[...] + cur


def _fin_mm_stats(acc, h2, es2, ed2, c, b, wT, wb):
    m = acc.shape[0]
    return pl.pallas_call(
        _fin_mm_stats_kernel,
        grid=(m // MB,),
        in_specs=[
            pl.BlockSpec((MB, DP), lambda i: (i, 0)),
            pl.BlockSpec((MB, DP), lambda i: (i, 0)),
            pl.BlockSpec((MB, 1), lambda i: (i, 0)),
            pl.BlockSpec((MB, 1), lambda i: (i, 0)),
            pl.BlockSpec((1, 1), lambda i: (0, 0)),
            pl.BlockSpec((1, D), lambda i: (0, 0)),
            pl.BlockSpec((D, D), lambda i: (0, 0)),
            pl.BlockSpec((1, D), lambda i: (0, 0)),
        ],
        out_specs=[
            pl.BlockSpec((MB, D), lambda i: (i, 0)),
            pl.BlockSpec((2, D), lambda i: (0, 0)),
        ],
        out_shape=[
            jax.ShapeDtypeStruct((m, D), jnp.float32),
            jax.ShapeDtypeStruct((2, D), jnp.float32),
        ],
    )(acc, h2, es2, ed2, c, b, wT, wb)


def _bn_mm_stats_kernel(z_ref, st_ref, g_ref, be_ref, w_ref, wb_ref,
                        z2_ref, st2_ref):
    mu = st_ref[0:1, :] * (1.0 / N)
    var = st_ref[1:2, :] * (1.0 / N) - mu * mu
    xn = g_ref[...] * (z_ref[...] - mu) * lax.rsqrt(var + 1e-5) + be_ref[...]
    h = _elu(xn)
    z2 = jnp.dot(h, w_ref[...], preferred_element_type=jnp.float32) + wb_ref[...]
    z2_ref[...] = z2
    cur = jnp.concatenate([jnp.sum(z2, axis=0, keepdims=True),
                           jnp.sum(z2 * z2, axis=0, keepdims=True)], axis=0)

    @pl.when(pl.program_id(0) == 0)
    def _():
        st2_ref[...] = cur

    @pl.when(pl.program_id(0) != 0)
    def _():
        st2_ref[...] = st2_ref[...] + cur


def _bn_mm_stats(z, st, g, be, wT, wb):
    m = z.shape[0]
    return pl.pallas_call(
        _bn_mm_stats_kernel,
        grid=(m // MB,),
        in_specs=[
            pl.BlockSpec((MB, D), lambda i: (i, 0)),
            pl.BlockSpec((2, D), lambda i: (0, 0)),
            pl.BlockSpec((1, D), lambda i: (0, 0)),
            pl.BlockSpec((1, D), lambda i: (0, 0)),
            pl.BlockSpec((D, D), lambda i: (0, 0)),
            pl.BlockSpec((1, D), lambda i: (0, 0)),
        ],
        out_specs=[
            pl.BlockSpec((MB, D), lambda i: (i, 0)),
            pl.BlockSpec((2, D), lambda i: (0, 0)),
        ],
        out_shape=[
            jax.ShapeDtypeStruct((m, D), jnp.float32),
            jax.ShapeDtypeStruct((2, D), jnp.float32),
        ],
    )(z, st, g, be, wT, wb)


def _bn_elu_kernel(z_ref, st_ref, g_ref, be_ref, o_ref):
    mu = st_ref[0:1, :] * (1.0 / N)
    var = st_ref[1:2, :] * (1.0 / N) - mu * mu
    xn = g_ref[...] * (z_ref[...] - mu) * lax.rsqrt(var + 1e-5) + be_ref[...]
    o_ref[...] = _elu(xn)


def _bn_elu(z, st, g, be):
    m = z.shape[0]
    return pl.pallas_call(
        _bn_elu_kernel,
        grid=(m // MB,),
        in_specs=[
            pl.BlockSpec((MB, D), lambda i: (i, 0)),
            pl.BlockSpec((2, D), lambda i: (0, 0)),
            pl.BlockSpec((1, D), lambda i: (0, 0)),
            pl.BlockSpec((1, D), lambda i: (0, 0)),
        ],
        out_specs=pl.BlockSpec((MB, D), lambda i: (i, 0)),
        out_shape=jax.ShapeDtypeStruct((m, D), jnp.float32),
    )(z, st, g, be)


def _sc_body(h_hbm, src_hbm, dst_hbm, es_hbm, ed_hbm, c_hbm, out_hbm,
             es_v, ed_v, seg_src, seg_dst, srcb, dstb,
             r0, r1, r2, r3, i0, i1, i2, i3,
             cv, ee_v, acc_sh, g0sem, g1sem, g2sem, g3sem,
             s0sem, s1sem, s2sem, s3sem):
    c = lax.axis_index("c")
    s = lax.axis_index("s")
    lo = c * HALF
    rbufs = (r0, r1, r2, r3)
    ibufs = (i0, i1, i2, i3)
    gsems = (g0sem, g1sem, g2sem, g3sem)
    ssems = (s0sem, s1sem, s2sem, s3sem)

    pltpu.sync_copy(es_hbm, es_v)
    pltpu.sync_copy(ed_hbm, ed_v)
    pltpu.sync_copy(c_hbm, cv)

    @pl.loop(0, 16)
    def _(i):
        for j in range(DP // 16):
            r0[i, pl.ds(j * 16, 16)] = jnp.zeros((16,), jnp.float32)

    @pl.loop(0, RPT, step=16)
    def _(r):
        pltpu.sync_copy(r0, acc_sh.at[pl.ds(s * RPT + r, 16)])

    plsc.subcore_barrier()

    cshift = cv[...]
    lanes = lax.iota(jnp.int32, 16)

    def scale(g, sv, dv, rbuf, cnt):
        a = plsc.load_gather(es_v, [sv])
        b = plsc.load_gather(ed_v, [dv])
        z = a + b
        e = jnp.where(z > 0, z, z * 0.2)
        ee = jnp.exp(e - cshift)
        ee = jnp.where(g * 16 + lanes < cnt, ee, 0.0)
        ee_v[...] = ee
        for i in range(16):
            bc = plsc.load_gather(ee_v, [jnp.full((16,), i, jnp.int32)])
            for j in range(DP // 16):
                sl = pl.ds(j * 16, 16)
                rbuf[i, sl] = rbuf[i, sl] * bc

    @pl.loop(0, NSEG)
    def _(seg):
        base = s * CHUNK + seg * SEG
        pltpu.sync_copy(src_hbm.at[pl.ds(base, SEG)], seg_src)
        pltpu.sync_copy(dst_hbm.at[pl.ds(base, SEG)], seg_dst)

        def scan_body(g, cnt):
            sv = seg_src[pl.ds(g * 16, 16)]
            dv = seg_dst[pl.ds(g * 16, 16)]
            m = (dv >= lo) & (dv < lo + HALF)
            plsc.store_compressed(srcb.at[pl.ds(cnt, 16)], sv, mask=m)
            plsc.store_compressed(dstb.at[pl.ds(cnt, 16)], dv, mask=m)
            pc = plsc.all_reduce_population_count(m)
            return cnt + jnp.max(pc)

        cnt = lax.fori_loop(0, SEG // 16, scan_body, jnp.int32(0))

        for t in range(4):
            srcb[pl.ds(cnt + 16 * t, 16)] = jnp.zeros((16,), jnp.int32)
            dstb[pl.ds(cnt + 16 * t, 16)] = jnp.full((16,), lo, jnp.int32)

        ng = (cnt + 15) // 16
        nquad = (ng + 3) // 4
        gmax = 4 * nquad

        def issue_gather(g, b):
            @pl.when(g < gmax)
            def _():
                sv = srcb[pl.ds(g * 16, 16)]
                pltpu.async_copy(h_hbm.at[sv], rbufs[b], gsems[b])

        def wait_gather(g, b):
            sv = srcb[pl.ds(g * 16, 16)]
            pltpu.make_async_copy(h_hbm.at[sv], rbufs[b], gsems[b]).wait()

        for b in range(4):
            issue_gather(jnp.int32(b), b)

        def quad_body(q, carry):
            a0 = 4 * q
            for b in range(4):
                g = a0 + b
                sv = srcb[pl.ds(g * 16, 16)]
                dv = dstb[pl.ds(g * 16, 16)]
                wait_gather(g, b)
                scale(g, sv, dv, rbufs[b], cnt)
                ibufs[b][...] = dv - lo
                pltpu.async_copy(rbufs[b], acc_sh.at[ibufs[b]], ssems[b],
                                 add=True)
                if b == 1:
                    for bb in range(2):
                        pltpu.make_async_copy(rbufs[bb], acc_sh.at[ibufs[bb]],
                                              ssems[bb]).wait()
                        issue_gather(a0 + 4 + bb, bb)
            for bb in range(2, 4):
                pltpu.make_async_copy(rbufs[bb], acc_sh.at[ibufs[bb]],
                                      ssems[bb]).wait()
                issue_gather(a0 + 4 + bb, bb)
            return carry

        lax.fori_loop(0, nquad, quad_body, jnp.int32(0))

    plsc.subcore_barrier()

    pltpu.sync_copy(acc_sh.at[pl.ds(s * RPT, RPT)],
                    out_hbm.at[pl.ds(c * ACC_ROWS + s * RPT, RPT)])


def _sc_gat_edges(h_pad, src, dst, es, ed, cvec):
    cp = pltpu.CompilerParams(needs_layout_passes=False,
                              use_tc_tiling_on_sc=False)
    mesh = plsc.VectorSubcoreMesh(core_axis_name="c", subcore_axis_name="s")
    fn = pl.kernel(
        _sc_body,
        compiler_params=cp,
        out_type=jax.ShapeDtypeStruct((2 * ACC_ROWS, DP), jnp.float32),
        mesh=mesh,
        scratch_types=[
            pltpu.VMEM((N,), jnp.float32),
            pltpu.VMEM((N,), jnp.float32),
            pltpu.VMEM((SEG,), jnp.int32),
            pltpu.VMEM((SEG,), jnp.int32),
            pltpu.VMEM((SEG + 64,), jnp.int32),
            pltpu.VMEM((SEG + 64,), jnp.int32),
            pltpu.VMEM((16, DP), jnp.float32),
            pltpu.VMEM((16, DP), jnp.float32),
            pltpu.VMEM((16, DP), jnp.float32),
            pltpu.VMEM((16, DP), jnp.float32),
            pltpu.VMEM((16,), jnp.int32),
            pltpu.VMEM((16,), jnp.int32),
            pltpu.VMEM((16,), jnp.int32),
            pltpu.VMEM((16,), jnp.int32),
            pltpu.VMEM((16,), jnp.float32),
            pltpu.VMEM((16,), jnp.float32),
            pltpu.VMEM_SHARED((ACC_ROWS, DP), jnp.float32),
            pltpu.SemaphoreType.DMA,
            pltpu.SemaphoreType.DMA,
            pltpu.SemaphoreType.DMA,
            pltpu.SemaphoreType.DMA,
            pltpu.SemaphoreType.DMA,
            pltpu.SemaphoreType.DMA,
            pltpu.SemaphoreType.DMA,
            pltpu.SemaphoreType.DMA,
        ],
    )
    return fn(h_pad, src, dst, es, ed, cvec)


def _halves(out):
    return jnp.concatenate([out[:HALF], out[ACC_ROWS:ACC_ROWS + HALF]], axis=0)


def kernel(x, edges, W1, a_src1, a_dst1, b1, W2, a_src2, a_dst2, b2,
           l1_w, l1_b, l2_w, l2_b, g1, be1, g2, be2):
    src = edges[0]
    dst = edges[1]
    w1tp = jnp.pad(W1.T, ((0, 0), (0, DP - D)))
    w2tp = jnp.pad(W2.T, ((0, 0), (0, DP - D)))

    h1, es1, ed1, m1 = _mm_att(x, w1tp, a_src1.reshape(1, D),
                               a_dst1.reshape(1, D))
    c1 = jnp.maximum(m1[0, 0] + m1[0, 1], 0.0)
    out1 = _sc_gat_edges(h1, src, dst, es1.reshape(N), ed1.reshape(N),
                         jnp.full((16,), c1, jnp.float32))
    acc1 = _halves(out1)

    h2, es2, ed2, m2 = _fin_mm_att(acc1, h1, es1, ed1, c1.reshape(1, 1),
                                   b1.reshape(1, D), w2tp,
                                   a_src2.reshape(1, D), a_dst2.reshape(1, D))
    c2 = jnp.maximum(m2[0, 0] + m2[0, 1], 0.0)
    out2 = _sc_gat_edges(h2, src, dst, es2.reshape(N), ed2.reshape(N),
                         jnp.full((16,), c2, jnp.float32))
    acc2 = _halves(out2)

    z1, st1 = _fin_mm_stats(acc2, h2, es2, ed2, c2.reshape(1, 1),
                            b2.reshape(1, D), l1_w.T, l1_b.reshape(1, D))
    z2, st2 = _bn_mm_stats(z1, st1, g1.reshape(1, D), be1.reshape(1, D),
                           l2_w.T, l2_b.reshape(1, D))
    return _bn_elu(z2, st2, g2.reshape(1, D), be2.reshape(1, D))


# speedup vs baseline: 1.7193x; 1.7193x over previous
import functools

import jax
import jax.numpy as jnp
from jax import lax
from jax.experimental import pallas as pl
from jax.experimental.pallas import tpu as pltpu
from jax.experimental.pallas import tpu_sc as plsc

N = 10000
D = 256
DP = 272
E = 320000
NCHUNK = 16
CHUNK = E // NCHUNK
NSEG = 10
SEG = CHUNK // NSEG
HALF = N // 2
RPT = 320
ACC_ROWS = 16 * RPT
MB = 400


def _mm_att_kernel(x_ref, w_ref, asrc_ref, adst_ref, h_ref, es_ref, ed_ref,
                   m_ref):
    mm = jnp.dot(x_ref[...], w_ref[...], preferred_element_type=jnp.float32)
    h_ref[...] = mm
    h_ref[:, D:] = jnp.ones((MB, DP - D), jnp.float32)
    hv = mm[:, :D]
    es = jnp.sum(hv * asrc_ref[...], axis=1, keepdims=True)
    ed = jnp.sum(hv * adst_ref[...], axis=1, keepdims=True)
    es_ref[...] = es
    ed_ref[...] = ed
    cur = jnp.concatenate([jnp.max(es).reshape(1, 1),
                           jnp.max(ed).reshape(1, 1)], axis=1)

    @pl.when(pl.program_id(0) == 0)
    def _():
        m_ref[...] = cur

    @pl.when(pl.program_id(0) != 0)
    def _():
        m_ref[...] = jnp.maximum(m_ref[...], cur)


def _mm_att(x, wT_pad, a_src, a_dst):
    m, k = x.shape
    return pl.pallas_call(
        _mm_att_kernel,
        grid=(m // MB,),
        in_specs=[
            pl.BlockSpec((MB, k), lambda i: (i, 0)),
            pl.BlockSpec((k, DP), lambda i: (0, 0)),
            pl.BlockSpec((1, D), lambda i: (0, 0)),
            pl.BlockSpec((1, D), lambda i: (0, 0)),
        ],
        out_specs=[
            pl.BlockSpec((MB, DP), lambda i: (i, 0)),
            pl.BlockSpec((MB, 1), lambda i: (i, 0)),
            pl.BlockSpec((MB, 1), lambda i: (i, 0)),
            pl.BlockSpec((1, 2), lambda i: (0, 0)),
        ],
        out_shape=[
            jax.ShapeDtypeStruct((m, DP), jnp.float32),
            jax.ShapeDtypeStruct((m, 1), jnp.float32),
            jax.ShapeDtypeStruct((m, 1), jnp.float32),
            jax.ShapeDtypeStruct((1, 2), jnp.float32),
        ],
    )(x, wT_pad, a_src, a_dst)


def _elu(x):
    return jnp.where(x > 0, x, jnp.exp(jnp.minimum(x, 0.0)) - 1.0)


def _self_loop_finish(acc_ref, h_ref, es_ref, ed_ref, c_ref, b_ref):
    z = es_ref[...] + ed_ref[...]
    e = jnp.where(z > 0, z, z * 0.2)
    eeself = jnp.exp(e - c_ref[0, 0])
    accf = acc_ref[...] + eeself * h_ref[...]
    return accf[:, :D] / accf[:, D:D + 1] + b_ref[...]


def _fin_mm_att_kernel(acc_ref, h1_ref, es1_ref, ed1_ref, c_ref, b_ref,
                       w_ref, asrc_ref, adst_ref,
                       h_ref, es_ref, ed_ref, m_ref):
    hprev = _elu(_self_loop_finish(acc_ref, h1_ref, es1_ref, ed1_ref,
                                   c_ref, b_ref))
    mm = jnp.dot(hprev, w_ref[...], preferred_element_type=jnp.float32)
    h_ref[...] = mm
    h_ref[:, D:] = jnp.ones((MB, DP - D), jnp.float32)
    hv = mm[:, :D]
    es = jnp.sum(hv * asrc_ref[...], axis=1, keepdims=True)
    ed = jnp.sum(hv * adst_ref[...], axis=1, keepdims=True)
    es_ref[...] = es
    ed_ref[...] = ed
    cur = jnp.concatenate([jnp.max(es).reshape(1, 1),
                           jnp.max(ed).reshape(1, 1)], axis=1)

    @pl.when(pl.program_id(0) == 0)
    def _():
        m_ref[...] = cur

    @pl.when(pl.program_id(0) != 0)
    def _():
        m_ref[...] = jnp.maximum(m_ref[...], cur)


def _fin_mm_att(acc, h1, es1, ed1, c, b, wT_pad, a_src, a_dst):
    m = acc.shape[0]
    return pl.pallas_call(
        _fin_mm_att_kernel,
        grid=(m // MB,),
        in_specs=[
            pl.BlockSpec((MB, DP), lambda i: (i, 0)),
            pl.BlockSpec((MB, DP), lambda i: (i, 0)),
            pl.BlockSpec((MB, 1), lambda i: (i, 0)),
            pl.BlockSpec((MB, 1), lambda i: (i, 0)),
            pl.BlockSpec((1, 1), lambda i: (0, 0)),
            pl.BlockSpec((1, D), lambda i: (0, 0)),
            pl.BlockSpec((D, DP), lambda i: (0, 0)),
            pl.BlockSpec((1, D), lambda i: (0, 0)),
            pl.BlockSpec((1, D), lambda i: (0, 0)),
        ],
        out_specs=[
            pl.BlockSpec((MB, DP), lambda i: (i, 0)),
            pl.BlockSpec((MB, 1), lambda i: (i, 0)),
            pl.BlockSpec((MB, 1), lambda i: (i, 0)),
            pl.BlockSpec((1, 2), lambda i: (0, 0)),
        ],
        out_shape=[
            jax.ShapeDtypeStruct((m, DP), jnp.float32),
            jax.ShapeDtypeStruct((m, 1), jnp.float32),
            jax.ShapeDtypeStruct((m, 1), jnp.float32),
            jax.ShapeDtypeStruct((1, 2), jnp.float32),
        ],
    )(acc, h1, es1, ed1, c, b, wT_pad, a_src, a_dst)


def _fin_mm_stats_kernel(acc_ref, h2_ref, es2_ref, ed2_ref, c_ref, b_ref,
                         w_ref, wb_ref, z_ref, st_ref):
    h = _elu(_self_loop_finish(acc_ref, h2_ref, es2_ref, ed2_ref,
                               c_ref, b_ref))
    z = jnp.dot(h, w_ref[...], preferred_element_type=jnp.float32) + wb_ref[...]
    z_ref[...] = z
    cur = jnp.concatenate([jnp.sum(z, axis=0, keepdims=True),
                           jnp.sum(z * z, axis=0, keepdims=True)], axis=0)

    @pl.when(pl.program_id(0) == 0)
    def _():
        st_ref[...] = cur

    @pl.when(pl.program_id(0) != 0)
    def _():
        st_ref[...] = st_ref[...] + cur


def _fin_mm_stats(acc, h2, es2, ed2, c, b, wT, wb):
    m = acc.shape[0]
    return pl.pallas_call(
        _fin_mm_stats_kernel,
        grid=(m // MB,),
        in_specs=[
            pl.BlockSpec((MB, DP), lambda i: (i, 0)),
            pl.BlockSpec((MB, DP), lambda i: (i, 0)),
            pl.BlockSpec((MB, 1), lambda i: (i, 0)),
            pl.BlockSpec((MB, 1), lambda i: (i, 0)),
            pl.BlockSpec((1, 1), lambda i: (0, 0)),
            pl.BlockSpec((1, D), lambda i: (0, 0)),
            pl.BlockSpec((D, D), lambda i: (0, 0)),
            pl.BlockSpec((1, D), lambda i: (0, 0)),
        ],
        out_specs=[
            pl.BlockSpec((MB, D), lambda i: (i, 0)),
            pl.BlockSpec((2, D), lambda i: (0, 0)),
        ],
        out_shape=[
            jax.ShapeDtypeStruct((m, D), jnp.float32),
            jax.ShapeDtypeStruct((2, D), jnp.float32),
        ],
    )(acc, h2, es2, ed2, c, b, wT, wb)


def _bn_mm_stats_kernel(z_ref, st_ref, g_ref, be_ref, w_ref, wb_ref,
                        z2_ref, st2_ref):
    mu = st_ref[0:1, :] * (1.0 / N)
    var = st_ref[1:2, :] * (1.0 / N) - mu * mu
    xn = g_ref[...] * (z_ref[...] - mu) * lax.rsqrt(var + 1e-5) + be_ref[...]
    h = _elu(xn)
    z2 = jnp.dot(h, w_ref[...], preferred_element_type=jnp.float32) + wb_ref[...]
    z2_ref[...] = z2
    cur = jnp.concatenate([jnp.sum(z2, axis=0, keepdims=True),
                           jnp.sum(z2 * z2, axis=0, keepdims=True)], axis=0)

    @pl.when(pl.program_id(0) == 0)
    def _():
        st2_ref[...] = cur

    @pl.when(pl.program_id(0) != 0)
    def _():
        st2_ref[...] = st2_ref[...] + cur


def _bn_mm_stats(z, st, g, be, wT, wb):
    m = z.shape[0]
    return pl.pallas_call(
        _bn_mm_stats_kernel,
        grid=(m // MB,),
        in_specs=[
            pl.BlockSpec((MB, D), lambda i: (i, 0)),
            pl.BlockSpec((2, D), lambda i: (0, 0)),
            pl.BlockSpec((1, D), lambda i: (0, 0)),
            pl.BlockSpec((1, D), lambda i: (0, 0)),
            pl.BlockSpec((D, D), lambda i: (0, 0)),
            pl.BlockSpec((1, D), lambda i: (0, 0)),
        ],
        out_specs=[
            pl.BlockSpec((MB, D), lambda i: (i, 0)),
            pl.BlockSpec((2, D), lambda i: (0, 0)),
        ],
        out_shape=[
            jax.ShapeDtypeStruct((m, D), jnp.float32),
            jax.ShapeDtypeStruct((2, D), jnp.float32),
        ],
    )(z, st, g, be, wT, wb)


def _bn_elu_kernel(z_ref, st_ref, g_ref, be_ref, o_ref):
    mu = st_ref[0:1, :] * (1.0 / N)
    var = st_ref[1:2, :] * (1.0 / N) - mu * mu
    xn = g_ref[...] * (z_ref[...] - mu) * lax.rsqrt(var + 1e-5) + be_ref[...]
    o_ref[...] = _elu(xn)


def _bn_elu(z, st, g, be):
    m = z.shape[0]
    return pl.pallas_call(
        _bn_elu_kernel,
        grid=(m // MB,),
        in_specs=[
            pl.BlockSpec((MB, D), lambda i: (i, 0)),
            pl.BlockSpec((2, D), lambda i: (0, 0)),
            pl.BlockSpec((1, D), lambda i: (0, 0)),
            pl.BlockSpec((1, D), lambda i: (0, 0)),
        ],
        out_specs=pl.BlockSpec((MB, D), lambda i: (i, 0)),
        out_shape=jax.ShapeDtypeStruct((m, D), jnp.float32),
    )(z, st, g, be)


def _sc_body(h_hbm, src_hbm, dst_hbm, es_hbm, ed_hbm, c_hbm, out_hbm,
             es_v, ed_v, seg_src, seg_dst, srcb, dstb,
             r0, r1, r2, r3, i0, i1, i2, i3,
             cv, ee_v, acc_sh, g0sem, g1sem, g2sem, g3sem,
             s0sem, s1sem, s2sem, s3sem):
    c = lax.axis_index("c")
    s = lax.axis_index("s")
    lo = c * HALF
    rbufs = (r0, r1, r2, r3)
    ibufs = (i0, i1, i2, i3)
    gsems = (g0sem, g1sem, g2sem, g3sem)
    ssems = (s0sem, s1sem, s2sem, s3sem)

    pltpu.sync_copy(es_hbm, es_v)
    pltpu.sync_copy(ed_hbm.at[pl.ds(lo, HALF)], ed_v)
    pltpu.sync_copy(c_hbm, cv)

    @pl.loop(0, 16)
    def _(i):
        for j in range(DP // 16):
            r0[i, pl.ds(j * 16, 16)] = jnp.zeros((16,), jnp.float32)

    @pl.loop(0, RPT, step=16)
    def _(r):
        pltpu.sync_copy(r0, acc_sh.at[pl.ds(s * RPT + r, 16)])

    plsc.subcore_barrier()

    cshift = cv[...]
    lanes = lax.iota(jnp.int32, 16)

    def scale(g, sv, dvl, rbuf, cnt):
        a = plsc.load_gather(es_v, [sv])
        b = plsc.load_gather(ed_v, [dvl])
        z = a + b
        e = jnp.where(z > 0, z, z * 0.2)
        ee = jnp.exp(e - cshift)
        ee = jnp.where(g * 16 + lanes < cnt, ee, 0.0)
        ee_v[...] = ee
        for i in range(16):
            bc = plsc.load_gather(ee_v, [jnp.full((16,), i, jnp.int32)])
            for j in range(DP // 16):
                sl = pl.ds(j * 16, 16)
                rbuf[i, sl] = rbuf[i, sl] * bc

    @pl.loop(0, NSEG)
    def _(seg):
        base = s * CHUNK + seg * SEG
        pltpu.sync_copy(src_hbm.at[pl.ds(base, SEG)], seg_src)
        pltpu.sync_copy(dst_hbm.at[pl.ds(base, SEG)], seg_dst)

        def scan_body(g, cnt):
            sv = seg_src[pl.ds(g * 16, 16)]
            dv = seg_dst[pl.ds(g * 16, 16)]
            m = (dv >= lo) & (dv < lo + HALF)
            plsc.store_compressed(srcb.at[pl.ds(cnt, 16)], sv, mask=m)
            plsc.store_compressed(dstb.at[pl.ds(cnt, 16)], dv, mask=m)
            pc = plsc.all_reduce_population_count(m)
            return cnt + jnp.max(pc)

        cnt = lax.fori_loop(0, SEG // 16, scan_body, jnp.int32(0))

        for t in range(4):
            srcb[pl.ds(cnt + 16 * t, 16)] = jnp.zeros((16,), jnp.int32)
            dstb[pl.ds(cnt + 16 * t, 16)] = jnp.full((16,), lo, jnp.int32)

        ng = (cnt + 15) // 16
        nquad = (ng + 3) // 4

        def quad_body(q, carry):
            a0 = 4 * q
            svs = [srcb[pl.ds((a0 + b) * 16, 16)] for b in range(4)]
            dvs = [dstb[pl.ds((a0 + b) * 16, 16)] for b in range(4)]
            cgs = [pltpu.async_copy(h_hbm.at[svs[b]], rbufs[b], gsems[b])
                   for b in range(4)]
            css = []
            for b in range(4):
                cgs[b].wait()
                dvl = dvs[b] - lo
                scale(a0 + b, svs[b], dvl, rbufs[b], cnt)
                ibufs[b][...] = dvl
                css.append(pltpu.async_copy(rbufs[b], acc_sh.at[ibufs[b]],
                                            ssems[b], add=True))
            for cs in css:
                cs.wait()
            return carry

        lax.fori_loop(0, nquad, quad_body, jnp.int32(0))

    plsc.subcore_barrier()

    pltpu.sync_copy(acc_sh.at[pl.ds(s * RPT, RPT)],
                    out_hbm.at[pl.ds(c * ACC_ROWS + s * RPT, RPT)])


def _sc_gat_edges(h_pad, src, dst, es, ed, cvec):
    cp = pltpu.CompilerParams(needs_layout_passes=False,
                              use_tc_tiling_on_sc=False)
    mesh = plsc.VectorSubcoreMesh(core_axis_name="c", subcore_axis_name="s")
    fn = pl.kernel(
        _sc_body,
        compiler_params=cp,
        out_type=jax.ShapeDtypeStruct((2 * ACC_ROWS, DP), jnp.float32),
        mesh=mesh,
        scratch_types=[
            pltpu.VMEM((N,), jnp.float32),
            pltpu.VMEM((HALF,), jnp.float32),
            pltpu.VMEM((SEG,), jnp.int32),
            pltpu.VMEM((SEG,), jnp.int32),
            pltpu.VMEM((SEG + 64,), jnp.int32),
            pltpu.VMEM((SEG + 64,), jnp.int32),
            pltpu.VMEM((16, DP), jnp.float32),
            pltpu.VMEM((16, DP), jnp.float32),
            pltpu.VMEM((16, DP), jnp.float32),
            pltpu.VMEM((16, DP), jnp.float32),
            pltpu.VMEM((16,), jnp.int32),
            pltpu.VMEM((16,), jnp.int32),
            pltpu.VMEM((16,), jnp.int32),
            pltpu.VMEM((16,), jnp.int32),
            pltpu.VMEM((16,), jnp.float32),
            pltpu.VMEM((16,), jnp.float32),
            pltpu.VMEM_SHARED((ACC_ROWS, DP), jnp.float32),
            pltpu.SemaphoreType.DMA,
            pltpu.SemaphoreType.DMA,
            pltpu.SemaphoreType.DMA,
            pltpu.SemaphoreType.DMA,
            pltpu.SemaphoreType.DMA,
            pltpu.SemaphoreType.DMA,
            pltpu.SemaphoreType.DMA,
            pltpu.SemaphoreType.DMA,
        ],
    )
    return fn(h_pad, src, dst, es, ed, cvec)


def _halves(out):
    return jnp.concatenate([out[:HALF], out[ACC_ROWS:ACC_ROWS + HALF]], axis=0)


def kernel(x, edges, W1, a_src1, a_dst1, b1, W2, a_src2, a_dst2, b2,
           l1_w, l1_b, l2_w, l2_b, g1, be1, g2, be2):
    src = edges[0]
    dst = edges[1]
    w1tp = jnp.pad(W1.T, ((0, 0), (0, DP - D)))
    w2tp = jnp.pad(W2.T, ((0, 0), (0, DP - D)))

    h1, es1, ed1, m1 = _mm_att(x, w1tp, a_src1.reshape(1, D),
                               a_dst1.reshape(1, D))
    c1 = jnp.maximum(m1[0, 0] + m1[0, 1], 0.0)
    out1 = _sc_gat_edges(h1, src, dst, es1.reshape(N), ed1.reshape(N),
                         jnp.full((16,), c1, jnp.float32))
    acc1 = _halves(out1)

    h2, es2, ed2, m2 = _fin_mm_att(acc1, h1, es1, ed1, c1.reshape(1, 1),
                                   b1.reshape(1, D), w2tp,
                                   a_src2.reshape(1, D), a_dst2.reshape(1, D))
    c2 = jnp.maximum(m2[0, 0] + m2[0, 1], 0.0)
    out2 = _sc_gat_edges(h2, src, dst, es2.reshape(N), ed2.reshape(N),
                         jnp.full((16,), c2, jnp.float32))
    acc2 = _halves(out2)

    z1, st1 = _fin_mm_stats(acc2, h2, es2, ed2, c2.reshape(1, 1),
                            b2.reshape(1, D), l1_w.T, l1_b.reshape(1, D))
    z2, st2 = _bn_mm_stats(z1, st1, g1.reshape(1, D), be1.reshape(1, D),
                           l2_w.T, l2_b.reshape(1, D))
    return _bn_elu(z2, st2, g2.reshape(1, D), be2.reshape(1, D))


# speedup vs baseline: 1.9363x; 1.1263x over previous
import functools

import jax
import jax.numpy as jnp
from jax import lax
from jax.experimental import pallas as pl
from jax.experimental.pallas import tpu as pltpu
from jax.experimental.pallas import tpu_sc as plsc

N = 10000
D = 256
DP = 272
E = 320000
NCHUNK = 16
CHUNK = E // NCHUNK
NSEG = 10
SEG = CHUNK // NSEG
HALF = N // 2
RPT = 320
ACC_ROWS = 16 * RPT
MB = 400


def _mm_att_kernel(x_ref, w_ref, asrc_ref, adst_ref, h_ref, es_ref, ed_ref,
                   m_ref):
    mm = jnp.dot(x_ref[...], w_ref[...], preferred_element_type=jnp.float32)
    h_ref[...] = mm
    h_ref[:, D:] = jnp.ones((MB, DP - D), jnp.float32)
    hv = mm[:, :D]
    es = jnp.sum(hv * asrc_ref[...], axis=1, keepdims=True)
    ed = jnp.sum(hv * adst_ref[...], axis=1, keepdims=True)
    es_ref[...] = es
    ed_ref[...] = ed
    cur = jnp.concatenate([jnp.max(es).reshape(1, 1),
                           jnp.max(ed).reshape(1, 1)], axis=1)

    @pl.when(pl.program_id(0) == 0)
    def _():
        m_ref[...] = cur

    @pl.when(pl.program_id(0) != 0)
    def _():
        m_ref[...] = jnp.maximum(m_ref[...], cur)


def _mm_att(x, wT_pad, a_src, a_dst):
    m, k = x.shape
    return pl.pallas_call(
        _mm_att_kernel,
        grid=(m // MB,),
        in_specs=[
            pl.BlockSpec((MB, k), lambda i: (i, 0)),
            pl.BlockSpec((k, DP), lambda i: (0, 0)),
            pl.BlockSpec((1, D), lambda i: (0, 0)),
            pl.BlockSpec((1, D), lambda i: (0, 0)),
        ],
        out_specs=[
            pl.BlockSpec((MB, DP), lambda i: (i, 0)),
            pl.BlockSpec((MB, 1), lambda i: (i, 0)),
            pl.BlockSpec((MB, 1), lambda i: (i, 0)),
            pl.BlockSpec((1, 2), lambda i: (0, 0)),
        ],
        out_shape=[
            jax.ShapeDtypeStruct((m, DP), jnp.float32),
            jax.ShapeDtypeStruct((m, 1), jnp.float32),
            jax.ShapeDtypeStruct((m, 1), jnp.float32),
            jax.ShapeDtypeStruct((1, 2), jnp.float32),
        ],
    )(x, wT_pad, a_src, a_dst)


def _elu(x):
    return jnp.where(x > 0, x, jnp.exp(jnp.minimum(x, 0.0)) - 1.0)


def _self_loop_finish(acc_ref, h_ref, es_ref, ed_ref, c_ref, b_ref):
    z = es_ref[...] + ed_ref[...]
    e = jnp.where(z > 0, z, z * 0.2)
    eeself = jnp.exp(e - c_ref[0, 0])
    accf = acc_ref[...] + eeself * h_ref[...]
    return accf[:, :D] / accf[:, D:D + 1] + b_ref[...]


def _fin_mm_att_kernel(acc_ref, h1_ref, es1_ref, ed1_ref, c_ref, b_ref,
                       w_ref, asrc_ref, adst_ref,
                       h_ref, es_ref, ed_ref, m_ref):
    hprev = _elu(_self_loop_finish(acc_ref, h1_ref, es1_ref, ed1_ref,
                                   c_ref, b_ref))
    mm = jnp.dot(hprev, w_ref[...], preferred_element_type=jnp.float32)
    h_ref[...] = mm
    h_ref[:, D:] = jnp.ones((MB, DP - D), jnp.float32)
    hv = mm[:, :D]
    es = jnp.sum(hv * asrc_ref[...], axis=1, keepdims=True)
    ed = jnp.sum(hv * adst_ref[...], axis=1, keepdims=True)
    es_ref[...] = es
    ed_ref[...] = ed
    cur = jnp.concatenate([jnp.max(es).reshape(1, 1),
                           jnp.max(ed).reshape(1, 1)], axis=1)

    @pl.when(pl.program_id(0) == 0)
    def _():
        m_ref[...] = cur

    @pl.when(pl.program_id(0) != 0)
    def _():
        m_ref[...] = jnp.maximum(m_ref[...], cur)


def _fin_mm_att(acc, h1, es1, ed1, c, b, wT_pad, a_src, a_dst):
    m = acc.shape[0]
    return pl.pallas_call(
        _fin_mm_att_kernel,
        grid=(m // MB,),
        in_specs=[
            pl.BlockSpec((MB, DP), lambda i: (i, 0)),
            pl.BlockSpec((MB, DP), lambda i: (i, 0)),
            pl.BlockSpec((MB, 1), lambda i: (i, 0)),
            pl.BlockSpec((MB, 1), lambda i: (i, 0)),
            pl.BlockSpec((1, 1), lambda i: (0, 0)),
            pl.BlockSpec((1, D), lambda i: (0, 0)),
            pl.BlockSpec((D, DP), lambda i: (0, 0)),
            pl.BlockSpec((1, D), lambda i: (0, 0)),
            pl.BlockSpec((1, D), lambda i: (0, 0)),
        ],
        out_specs=[
            pl.BlockSpec((MB, DP), lambda i: (i, 0)),
            pl.BlockSpec((MB, 1), lambda i: (i, 0)),
            pl.BlockSpec((MB, 1), lambda i: (i, 0)),
            pl.BlockSpec((1, 2), lambda i: (0, 0)),
        ],
        out_shape=[
            jax.ShapeDtypeStruct((m, DP), jnp.float32),
            jax.ShapeDtypeStruct((m, 1), jnp.float32),
            jax.ShapeDtypeStruct((m, 1), jnp.float32),
            jax.ShapeDtypeStruct((1, 2), jnp.float32),
        ],
    )(acc, h1, es1, ed1, c, b, wT_pad, a_src, a_dst)


def _fin_mm_stats_kernel(acc_ref, h2_ref, es2_ref, ed2_ref, c_ref, b_ref,
                         w_ref, wb_ref, z_ref, st_ref):
    h = _elu(_self_loop_finish(acc_ref, h2_ref, es2_ref, ed2_ref,
                               c_ref, b_ref))
    z = jnp.dot(h, w_ref[...], preferred_element_type=jnp.float32) + wb_ref[...]
    z_ref[...] = z
    cur = jnp.concatenate([jnp.sum(z, axis=0, keepdims=True),
                           jnp.sum(z * z, axis=0, keepdims=True)], axis=0)

    @pl.when(pl.program_id(0) == 0)
    def _():
        st_ref[...] = cur

    @pl.when(pl.program_id(0) != 0)
    def _():
        st_ref[...] = st_ref[...] + cur


def _fin_mm_stats(acc, h2, es2, ed2, c, b, wT, wb):
    m = acc.shape[0]
    return pl.pallas_call(
        _fin_mm_stats_kernel,
        grid=(m // MB,),
        in_specs=[
            pl.BlockSpec((MB, DP), lambda i: (i, 0)),
            pl.BlockSpec((MB, DP), lambda i: (i, 0)),
            pl.BlockSpec((MB, 1), lambda i: (i, 0)),
            pl.BlockSpec((MB, 1), lambda i: (i, 0)),
            pl.BlockSpec((1, 1), lambda i: (0, 0)),
            pl.BlockSpec((1, D), lambda i: (0, 0)),
            pl.BlockSpec((D, D), lambda i: (0, 0)),
            pl.BlockSpec((1, D), lambda i: (0, 0)),
        ],
        out_specs=[
            pl.BlockSpec((MB, D), lambda i: (i, 0)),
            pl.BlockSpec((2, D), lambda i: (0, 0)),
        ],
        out_shape=[
            jax.ShapeDtypeStruct((m, D), jnp.float32),
            jax.ShapeDtypeStruct((2, D), jnp.float32),
        ],
    )(acc, h2, es2, ed2, c, b, wT, wb)


def _bn_mm_stats_kernel(z_ref, st_ref, g_ref, be_ref, w_ref, wb_ref,
                        z2_ref, st2_ref):
    mu = st_ref[0:1, :] * (1.0 / N)
    var = st_ref[1:2, :] * (1.0 / N) - mu * mu
    xn = g_ref[...] * (z_ref[...] - mu) * lax.rsqrt(var + 1e-5) + be_ref[...]
    h = _elu(xn)
    z2 = jnp.dot(h, w_ref[...], preferred_element_type=jnp.float32) + wb_ref[...]
    z2_ref[...] = z2
    cur = jnp.concatenate([jnp.sum(z2, axis=0, keepdims=True),
                           jnp.sum(z2 * z2, axis=0, keepdims=True)], axis=0)

    @pl.when(pl.program_id(0) == 0)
    def _():
        st2_ref[...] = cur

    @pl.when(pl.program_id(0) != 0)
    def _():
        st2_ref[...] = st2_ref[...] + cur


def _bn_mm_stats(z, st, g, be, wT, wb):
    m = z.shape[0]
    return pl.pallas_call(
        _bn_mm_stats_kernel,
        grid=(m // MB,),
        in_specs=[
            pl.BlockSpec((MB, D), lambda i: (i, 0)),
            pl.BlockSpec((2, D), lambda i: (0, 0)),
            pl.BlockSpec((1, D), lambda i: (0, 0)),
            pl.BlockSpec((1, D), lambda i: (0, 0)),
            pl.BlockSpec((D, D), lambda i: (0, 0)),
            pl.BlockSpec((1, D), lambda i: (0, 0)),
        ],
        out_specs=[
            pl.BlockSpec((MB, D), lambda i: (i, 0)),
            pl.BlockSpec((2, D), lambda i: (0, 0)),
        ],
        out_shape=[
            jax.ShapeDtypeStruct((m, D), jnp.float32),
            jax.ShapeDtypeStruct((2, D), jnp.float32),
        ],
    )(z, st, g, be, wT, wb)


def _bn_elu_kernel(z_ref, st_ref, g_ref, be_ref, o_ref):
    mu = st_ref[0:1, :] * (1.0 / N)
    var = st_ref[1:2, :] * (1.0 / N) - mu * mu
    xn = g_ref[...] * (z_ref[...] - mu) * lax.rsqrt(var + 1e-5) + be_ref[...]
    o_ref[...] = _elu(xn)


def _bn_elu(z, st, g, be):
    m = z.shape[0]
    return pl.pallas_call(
        _bn_elu_kernel,
        grid=(m // MB,),
        in_specs=[
            pl.BlockSpec((MB, D), lambda i: (i, 0)),
            pl.BlockSpec((2, D), lambda i: (0, 0)),
            pl.BlockSpec((1, D), lambda i: (0, 0)),
            pl.BlockSpec((1, D), lambda i: (0, 0)),
        ],
        out_specs=pl.BlockSpec((MB, D), lambda i: (i, 0)),
        out_shape=jax.ShapeDtypeStruct((m, D), jnp.float32),
    )(z, st, g, be)


def _sc_body(h_hbm, src_hbm, dst_hbm, es_hbm, ed_hbm, c_hbm, out_hbm,
             es_v, ed_v, seg_src, seg_dst, srcb, dstb,
             r0, r1, i0, i1,
             cv, ee_v, acc_sh, g0sem, g1sem, s0sem, s1sem):
    c = lax.axis_index("c")
    s = lax.axis_index("s")
    lo = c * HALF
    pltpu.sync_copy(es_hbm, es_v)
    pltpu.sync_copy(ed_hbm.at[pl.ds(lo, HALF)], ed_v)
    pltpu.sync_copy(c_hbm, cv)

    @pl.loop(0, 16)
    def _(i):
        for j in range(DP // 16):
            r0[i, pl.ds(j * 16, 16)] = jnp.zeros((16,), jnp.float32)

    @pl.loop(0, RPT, step=16)
    def _(r):
        pltpu.sync_copy(r0, acc_sh.at[pl.ds(s * RPT + r, 16)])

    plsc.subcore_barrier()

    cshift = cv[...]
    lanes = lax.iota(jnp.int32, 16)

    def scale(g, sv, dvl, rbuf, cnt):
        a = plsc.load_gather(es_v, [sv])
        b = plsc.load_gather(ed_v, [dvl])
        z = a + b
        e = jnp.where(z > 0, z, z * 0.2)
        ee = jnp.exp(e - cshift)
        ee = jnp.where(g * 16 + lanes < cnt, ee, 0.0)
        ee_v[...] = ee
        for i in range(16):
            bc = plsc.load_gather(ee_v, [jnp.full((16,), i, jnp.int32)])
            for j in range(DP // 16):
                sl = pl.ds(j * 16, 16)
                rbuf[i, sl] = rbuf[i, sl] * bc

    @pl.loop(0, NSEG)
    def _(seg):
        base = s * CHUNK + seg * SEG
        pltpu.sync_copy(src_hbm.at[pl.ds(base, SEG)], seg_src)
        pltpu.sync_copy(dst_hbm.at[pl.ds(base, SEG)], seg_dst)

        def scan_body(g, cnt):
            sv = seg_src[pl.ds(g * 16, 16)]
            dv = seg_dst[pl.ds(g * 16, 16)]
            m = (dv >= lo) & (dv < lo + HALF)
            plsc.store_compressed(srcb.at[pl.ds(cnt, 16)], sv, mask=m)
            plsc.store_compressed(dstb.at[pl.ds(cnt, 16)], dv, mask=m)
            pc = plsc.all_reduce_population_count(m)
            return cnt + jnp.max(pc)

        cnt = lax.fori_loop(0, SEG // 16, scan_body, jnp.int32(0))

        for t in range(2):
            srcb[pl.ds(cnt + 16 * t, 16)] = jnp.zeros((16,), jnp.int32)
            dstb[pl.ds(cnt + 16 * t, 16)] = jnp.full((16,), lo, jnp.int32)

        ng = (cnt + 15) // 16
        npair = (ng + 1) // 2

        def pair_body(p, carry):
            g0 = 2 * p
            g1 = g0 + 1
            sv0 = srcb[pl.ds(g0 * 16, 16)]
            dv0 = dstb[pl.ds(g0 * 16, 16)]
            sv1 = srcb[pl.ds(g1 * 16, 16)]
            dv1 = dstb[pl.ds(g1 * 16, 16)]
            cg0 = pltpu.async_copy(h_hbm.at[sv0], r0, g0sem)
            cg1 = pltpu.async_copy(h_hbm.at[sv1], r1, g1sem)
            cg0.wait()
            dvl0 = dv0 - lo
            scale(g0, sv0, dvl0, r0, cnt)
            i0[...] = dvl0
            cs0 = pltpu.async_copy(r0, acc_sh.at[i0], s0sem, add=True)
            cg1.wait()
            dvl1 = dv1 - lo
            scale(g1, sv1, dvl1, r1, cnt)
            i1[...] = dvl1
            cs1 = pltpu.async_copy(r1, acc_sh.at[i1], s1sem, add=True)
            cs0.wait()
            cs1.wait()
            return carry

        lax.fori_loop(0, npair, pair_body, jnp.int32(0))

    plsc.subcore_barrier()

    pltpu.sync_copy(acc_sh.at[pl.ds(s * RPT, RPT)],
                    out_hbm.at[pl.ds(c * ACC_ROWS + s * RPT, RPT)])


def _sc_gat_edges(h_pad, src, dst, es, ed, cvec):
    cp = pltpu.CompilerParams(needs_layout_passes=False,
                              use_tc_tiling_on_sc=False)
    mesh = plsc.VectorSubcoreMesh(core_axis_name="c", subcore_axis_name="s")
    fn = pl.kernel(
        _sc_body,
        compiler_params=cp,
        out_type=jax.ShapeDtypeStruct((2 * ACC_ROWS, DP), jnp.float32),
        mesh=mesh,
        scratch_types=[
            pltpu.VMEM((N,), jnp.float32),
            pltpu.VMEM((HALF,), jnp.float32),
            pltpu.VMEM((SEG,), jnp.int32),
            pltpu.VMEM((SEG,), jnp.int32),
            pltpu.VMEM((SEG + 64,), jnp.int32),
            pltpu.VMEM((SEG + 64,), jnp.int32),
            pltpu.VMEM((16, DP), jnp.float32),
            pltpu.VMEM((16, DP), jnp.float32),
            pltpu.VMEM((16,), jnp.int32),
            pltpu.VMEM((16,), jnp.int32),
            pltpu.VMEM((16,), jnp.float32),
            pltpu.VMEM((16,), jnp.float32),
            pltpu.VMEM_SHARED((ACC_ROWS, DP), jnp.float32),
            pltpu.SemaphoreType.DMA,
            pltpu.SemaphoreType.DMA,
            pltpu.SemaphoreType.DMA,
            pltpu.SemaphoreType.DMA,
        ],
    )
    return fn(h_pad, src, dst, es, ed, cvec)


def _halves(out):
    return jnp.concatenate([out[:HALF], out[ACC_ROWS:ACC_ROWS + HALF]], axis=0)


def kernel(x, edges, W1, a_src1, a_dst1, b1, W2, a_src2, a_dst2, b2,
           l1_w, l1_b, l2_w, l2_b, g1, be1, g2, be2):
    src = edges[0]
    dst = edges[1]
    w1tp = jnp.pad(W1.T, ((0, 0), (0, DP - D)))
    w2tp = jnp.pad(W2.T, ((0, 0), (0, DP - D)))

    h1, es1, ed1, m1 = _mm_att(x, w1tp, a_src1.reshape(1, D),
                               a_dst1.reshape(1, D))
    c1 = jnp.maximum(m1[0, 0] + m1[0, 1], 0.0)
    out1 = _sc_gat_edges(h1, src, dst, es1.reshape(N), ed1.reshape(N),
                         jnp.full((16,), c1, jnp.float32))
    acc1 = _halves(out1)

    h2, es2, ed2, m2 = _fin_mm_att(acc1, h1, es1, ed1, c1.reshape(1, 1),
                                   b1.reshape(1, D), w2tp,
                                   a_src2.reshape(1, D), a_dst2.reshape(1, D))
    c2 = jnp.maximum(m2[0, 0] + m2[0, 1], 0.0)
    out2 = _sc_gat_edges(h2, src, dst, es2.reshape(N), ed2.reshape(N),
                         jnp.full((16,), c2, jnp.float32))
    acc2 = _halves(out2)

    z1, st1 = _fin_mm_stats(acc2, h2, es2, ed2, c2.reshape(1, 1),
                            b2.reshape(1, D), l1_w.T, l1_b.reshape(1, D))
    z2, st2 = _bn_mm_stats(z1, st1, g1.reshape(1, D), be1.reshape(1, D),
                           l2_w.T, l2_b.reshape(1, D))
    return _bn_elu(z2, st2, g2.reshape(1, D), be2.reshape(1, D))
